# baseline (speedup 1.0000x reference)
import functools

import jax
import jax.numpy as jnp
from jax import lax
from jax.experimental import pallas as pl
from jax.experimental.pallas import tpu as pltpu
from jax.experimental.pallas import tpu_sc as plsc

B = 4096
S = 200
D = 64
N_LOCS = 128
V = 1000000

SLOT = 1 << 18
TAIL = V - 488 * 2048

NC = 2
NS = 16
L = 16
NW = NC * NS
BPW = B // NW

SA = 128
SB = S - SA

TW = 2048

_mesh = plsc.VectorSubcoreMesh(core_axis_name="c", subcore_axis_name="s")


def _round_bf16_bits(f):
    b = lax.bitcast_convert_type(f, jnp.int32)
    return b + 0x7FFF + (lax.shift_right_logical(b, 16) & 1)


def _pack(t_ref):
    lo = _round_bf16_bits(t_ref[pl.ds(0, D // 2), :])
    hi = _round_bf16_bits(t_ref[pl.ds(D // 2, D // 2), :])
    w = (hi & jnp.int32(-65536)) | lax.shift_right_logical(lo, 16)
    return lax.bitcast_convert_type(w, jnp.float32).T


def _tr_body(t0_ref, t1_ref, t2_ref, t3_ref, tl_ref, o_ref):
    i = pl.program_id(0)
    o_ref[...] = jnp.concatenate(
        [_pack(t0_ref), _pack(t1_ref), _pack(t2_ref), _pack(t3_ref)], axis=1
    )

    @pl.when(i == (V - TAIL - 3 * SLOT) // TW)
    def _():
        o_ref[pl.ds(0, TAIL), pl.ds(3 * (D // 2), D // 2)] = _pack(tl_ref)


def _in_spec(q):
    return pl.BlockSpec(
        (D, TW),
        lambda i, q=q: (0, jnp.minimum(i + q * (SLOT // TW), V // TW - 1)),
    )


_transposer = pl.pallas_call(
    _tr_body,
    grid=(SLOT // TW,),
    in_specs=[_in_spec(0), _in_spec(1), _in_spec(2), _in_spec(3),
              pl.BlockSpec((D, TAIL), lambda i: (0, 0))],
    out_specs=pl.BlockSpec((TW, 2 * D), lambda i: (i, 0)),
    out_shape=jax.ShapeDtypeStruct((SLOT, 2 * D), jnp.float32),
)


@functools.partial(
    pl.kernel,
    mesh=_mesh,
    compiler_params=pltpu.CompilerParams(use_tc_tiling_on_sc=False),
    out_type=jax.ShapeDtypeStruct((B, D), jnp.bfloat16),
    scratch_types=[
        pltpu.VMEM((BPW, SA), jnp.int32),
        pltpu.VMEM((BPW, SB), jnp.int32),
        [pltpu.VMEM((SA, D), jnp.bfloat16)] * 4,
        [pltpu.VMEM((SB, D), jnp.bfloat16)] * 4,
        pltpu.VMEM((BPW, D), jnp.bfloat16),
        [pltpu.SemaphoreType.DMA] * 4,
        [pltpu.SemaphoreType.DMA] * 4,
    ],
)
def _pool_sc(xa_hbm, xb_hbm, emb_hbm, out_hbm,
             idx_a, idx_b, bufs_a, bufs_b, out_v, sems_a, sems_b):
    NSLOT = 4
    wid = lax.axis_index("s") * NC + lax.axis_index("c")
    base = wid * BPW

    pltpu.sync_copy(xa_hbm.at[pl.ds(base, BPW)], idx_a)
    pltpu.sync_copy(xb_hbm.at[pl.ds(base, BPW)], idx_b)

    def issue(i, s):
        pltpu.async_copy(emb_hbm.at[idx_a.at[i]], bufs_a[s], sems_a[s])
        pltpu.async_copy(emb_hbm.at[idx_b.at[i]], bufs_b[s], sems_b[s])

    def wait(s):
        pltpu.make_async_copy(emb_hbm.at[idx_a.at[0]], bufs_a[s], sems_a[s]).wait()
        pltpu.make_async_copy(emb_hbm.at[idx_b.at[0]], bufs_b[s], sems_b[s]).wait()

    def reduce_row(i, s):
        def red(buf):
            def body(j, ms):
                return tuple(
                    jnp.maximum(ms[c], buf[j, pl.ds(c * 2 * L, 2 * L)])
                    for c in range(2)
                )
            return body

        neg = jnp.full((2 * L,), -jnp.inf, jnp.bfloat16)
        ms = (neg, neg)
        ms = lax.fori_loop(0, SA, red(bufs_a[s]), ms, unroll=8)
        ms = lax.fori_loop(0, SB, red(bufs_b[s]), ms, unroll=8)
        for c in range(2):
            out_v[i, pl.ds(c * 2 * L, 2 * L)] = ms[c]

    for s in range(NSLOT - 1):
        issue(s, s)

    def group(p, carry):
        i0 = NSLOT * p
        for s in range(NSLOT):
            i = i0 + s
            issue(jnp.minimum(i + NSLOT - 1, BPW - 1), (s + NSLOT - 1) % NSLOT)
            wait(s)
            reduce_row(i, s)
        return carry

    lax.fori_loop(0, BPW // NSLOT, group, 0)
    for s in range(NSLOT - 1):
        wait(s)
    pltpu.sync_copy(out_v, out_hbm.at[pl.ds(base, BPW)])


def _mm_body(p_ref, w_ref, b_ref, o_ref):
    o_ref[...] = (
        lax.dot_general(
            p_ref[...].astype(jnp.float32), w_ref[...],
            (((1,), (1,)), ((), ())),
            preferred_element_type=jnp.float32,
        )
        + b_ref[...]
    )


_mm = pl.pallas_call(
    _mm_body,
    grid=(8,),
    in_specs=[
        pl.BlockSpec((B // 8, D), lambda i: (i, 0)),
        pl.BlockSpec((N_LOCS, D), lambda i: (0, 0)),
        pl.BlockSpec((1, N_LOCS), lambda i: (0, 0)),
    ],
    out_specs=pl.BlockSpec((B // 8, N_LOCS), lambda i: (i, 0)),
    out_shape=jax.ShapeDtypeStruct((B, N_LOCS), jnp.float32),
)


def kernel(x, emb, W_fc, b_fc):
    x = x.astype(jnp.int32)
    xr = ((x & (SLOT - 1)) << 2) | lax.shift_right_logical(x, 18)
    xa = xr[:, :SA]
    xb = xr[:, SA:]
    embT = emb.T
    tailT = lax.slice(embT, (0, V - TAIL), (D, V))
    pack_tab = _transposer(embT, embT, embT, embT, tailT)
    flat_tab = lax.bitcast_convert_type(
        pack_tab, jnp.bfloat16).reshape(4 * SLOT, D)
    pooled_i = _pool_sc(xa, xb, flat_tab)
    pooled = jnp.transpose(
        pooled_i.reshape(B, D // 2, 2), (0, 2, 1)).reshape(B, D)
    return _mm(pooled, W_fc, b_fc.reshape(1, N_LOCS))


# speedup vs baseline: 1.0023x; 1.0023x over previous
import functools

import jax
import jax.numpy as jnp
from jax import lax
from jax.experimental import pallas as pl
from jax.experimental.pallas import tpu as pltpu
from jax.experimental.pallas import tpu_sc as plsc

B = 4096
S = 200
D = 64
N_LOCS = 128
V = 1000000

SLOT = 1 << 18
TAIL = V - 488 * 2048

NC = 2
NS = 16
L = 16
NW = NC * NS
BPW = B // NW

SA = 128
SB = S - SA

TW = 2048

_mesh = plsc.VectorSubcoreMesh(core_axis_name="c", subcore_axis_name="s")


def _round_bf16_bits(f):
    b = lax.bitcast_convert_type(f, jnp.int32)
    return b + 0x7FFF + (lax.shift_right_logical(b, 16) & 1)


def _pack_cols(vt, q):
    lo = _round_bf16_bits(vt[:, 64 * q:64 * q + D // 2])
    hi = _round_bf16_bits(vt[:, 64 * q + D // 2:64 * (q + 1)])
    w = (hi & jnp.int32(-65536)) | lax.shift_right_logical(lo, 16)
    return lax.bitcast_convert_type(w, jnp.float32)


def _tr_body(t0_ref, t1_ref, t2_ref, t3_ref, tl_ref, o_ref):
    i = pl.program_id(0)
    vt = jnp.concatenate(
        [t0_ref[...], t1_ref[...], t2_ref[...], t3_ref[...]], axis=0
    ).T
    o_ref[...] = jnp.concatenate(
        [_pack_cols(vt, q) for q in range(4)], axis=1
    )

    @pl.when(i == (V - TAIL - 3 * SLOT) // TW)
    def _():
        t2 = jnp.concatenate([tl_ref[...], tl_ref[...]], axis=0).T
        o_ref[pl.ds(0, TAIL), pl.ds(3 * (D // 2), D // 2)] = _pack_cols(t2, 0)


def _in_spec(q):
    return pl.BlockSpec(
        (D, TW),
        lambda i, q=q: (0, jnp.minimum(i + q * (SLOT // TW), V // TW - 1)),
    )


_transposer = pl.pallas_call(
    _tr_body,
    grid=(SLOT // TW,),
    in_specs=[_in_spec(0), _in_spec(1), _in_spec(2), _in_spec(3),
              pl.BlockSpec((D, TAIL), lambda i: (0, 0))],
    out_specs=pl.BlockSpec((TW, 2 * D), lambda i: (i, 0)),
    out_shape=jax.ShapeDtypeStruct((SLOT, 2 * D), jnp.float32),
)


@functools.partial(
    pl.kernel,
    mesh=_mesh,
    compiler_params=pltpu.CompilerParams(use_tc_tiling_on_sc=False),
    out_type=jax.ShapeDtypeStruct((B, D), jnp.bfloat16),
    scratch_types=[
        pltpu.VMEM((BPW, SA), jnp.int32),
        pltpu.VMEM((BPW, SB), jnp.int32),
        [pltpu.VMEM((SA, D), jnp.bfloat16)] * 4,
        [pltpu.VMEM((SB, D), jnp.bfloat16)] * 4,
        pltpu.VMEM((BPW, D), jnp.bfloat16),
        [pltpu.SemaphoreType.DMA] * 4,
        [pltpu.SemaphoreType.DMA] * 4,
    ],
)
def _pool_sc(xa_hbm, xb_hbm, emb_hbm, out_hbm,
             idx_a, idx_b, bufs_a, bufs_b, out_v, sems_a, sems_b):
    NSLOT = 4
    wid = lax.axis_index("s") * NC + lax.axis_index("c")
    base = wid * BPW

    pltpu.sync_copy(xa_hbm.at[pl.ds(base, BPW)], idx_a)
    pltpu.sync_copy(xb_hbm.at[pl.ds(base, BPW)], idx_b)

    def issue(i, s):
        pltpu.async_copy(emb_hbm.at[idx_a.at[i]], bufs_a[s], sems_a[s])
        pltpu.async_copy(emb_hbm.at[idx_b.at[i]], bufs_b[s], sems_b[s])

    def wait(s):
        pltpu.make_async_copy(emb_hbm.at[idx_a.at[0]], bufs_a[s], sems_a[s]).wait()
        pltpu.make_async_copy(emb_hbm.at[idx_b.at[0]], bufs_b[s], sems_b[s]).wait()

    def reduce_row(i, s):
        def red(buf):
            def body(j, ms):
                return tuple(
                    jnp.maximum(ms[c], buf[j, pl.ds(c * 2 * L, 2 * L)])
                    for c in range(2)
                )
            return body

        neg = jnp.full((2 * L,), -jnp.inf, jnp.bfloat16)
        ms = (neg, neg)
        ms = lax.fori_loop(0, SA, red(bufs_a[s]), ms, unroll=8)
        ms = lax.fori_loop(0, SB, red(bufs_b[s]), ms, unroll=8)
        for c in range(2):
            out_v[i, pl.ds(c * 2 * L, 2 * L)] = ms[c]

    for s in range(NSLOT - 1):
        issue(s, s)

    def group(p, carry):
        i0 = NSLOT * p
        for s in range(NSLOT):
            i = i0 + s
            issue(jnp.minimum(i + NSLOT - 1, BPW - 1), (s + NSLOT - 1) % NSLOT)
            wait(s)
            reduce_row(i, s)
        return carry

    lax.fori_loop(0, BPW // NSLOT, group, 0)
    for s in range(NSLOT - 1):
        wait(s)
    pltpu.sync_copy(out_v, out_hbm.at[pl.ds(base, BPW)])


def _mm_body(p_ref, w_ref, b_ref, o_ref):
    o_ref[...] = (
        lax.dot_general(
            p_ref[...].astype(jnp.float32), w_ref[...],
            (((1,), (1,)), ((), ())),
            preferred_element_type=jnp.float32,
        )
        + b_ref[...]
    )


_mm = pl.pallas_call(
    _mm_body,
    grid=(8,),
    in_specs=[
        pl.BlockSpec((B // 8, D), lambda i: (i, 0)),
        pl.BlockSpec((N_LOCS, D), lambda i: (0, 0)),
        pl.BlockSpec((1, N_LOCS), lambda i: (0, 0)),
    ],
    out_specs=pl.BlockSpec((B // 8, N_LOCS), lambda i: (i, 0)),
    out_shape=jax.ShapeDtypeStruct((B, N_LOCS), jnp.float32),
)


def kernel(x, emb, W_fc, b_fc):
    x = x.astype(jnp.int32)
    xr = ((x & (SLOT - 1)) << 2) | lax.shift_right_logical(x, 18)
    xa = xr[:, :SA]
    xb = xr[:, SA:]
    embT = emb.T
    tailT = lax.slice(embT, (0, V - TAIL), (D, V))
    pack_tab = _transposer(embT, embT, embT, embT, tailT)
    flat_tab = lax.bitcast_convert_type(
        pack_tab, jnp.bfloat16).reshape(4 * SLOT, D)
    pooled_i = _pool_sc(xa, xb, flat_tab)
    pooled = jnp.transpose(
        pooled_i.reshape(B, D // 2, 2), (0, 2, 1)).reshape(B, D)
    return _mm(pooled, W_fc, b_fc.reshape(1, N_LOCS))


# speedup vs baseline: 78.0391x; 77.8615x over previous
import functools

import jax
import jax.numpy as jnp
from jax import lax
from jax.experimental import pallas as pl
from jax.experimental.pallas import tpu as pltpu
from jax.experimental.pallas import tpu_sc as plsc

B = 4096
S = 200
D = 64
N_LOCS = 128
V = 1000000

VP = 1 << 19
TAIL = V - 488 * 2048

NC = 2
NS = 16
L = 16
NW = NC * NS
BPW = B // NW

SA = 128
SB = S - SA

TW = 2048

_mesh = plsc.VectorSubcoreMesh(core_axis_name="c", subcore_axis_name="s")


def _tr_body(t1_ref, t2_ref, tl_ref, o_ref):
    i = pl.program_id(0)
    v = jnp.concatenate([t1_ref[...], t2_ref[...]], axis=0)
    o_ref[...] = v.T

    @pl.when(i == (V - TAIL - VP) // TW)
    def _():
        o_ref[pl.ds(0, TAIL), pl.ds(D, D)] = tl_ref[...].T


_transposer = pl.pallas_call(
    _tr_body,
    grid=(VP // TW,),
    in_specs=[
        pl.BlockSpec((D, TW), lambda i: (0, i)),
        pl.BlockSpec((D, TW), lambda i: (0, jnp.minimum(i + VP // TW,
                                                        V // TW - 1))),
        pl.BlockSpec((D, TAIL), lambda i: (0, 0)),
    ],
    out_specs=pl.BlockSpec((TW, 2 * D), lambda i: (i, 0)),
    out_shape=jax.ShapeDtypeStruct((VP, 2 * D), jnp.float32),
)


@functools.partial(
    pl.kernel,
    mesh=_mesh,
    compiler_params=pltpu.CompilerParams(use_tc_tiling_on_sc=False),
    out_type=jax.ShapeDtypeStruct((B, D), jnp.float32),
    scratch_types=[
        pltpu.VMEM((BPW, SA), jnp.int32),
        pltpu.VMEM((BPW, SB), jnp.int32),
        [pltpu.VMEM((SA, D), jnp.float32)] * 4,
        [pltpu.VMEM((SB, D), jnp.float32)] * 4,
        pltpu.VMEM((BPW, D), jnp.float32),
        [pltpu.SemaphoreType.DMA] * 4,
        [pltpu.SemaphoreType.DMA] * 4,
    ],
)
def _pool_sc(xa_hbm, xb_hbm, emb_hbm, out_hbm,
             idx_a, idx_b, bufs_a, bufs_b, out_v, sems_a, sems_b):
    NSLOT = 4
    wid = lax.axis_index("s") * NC + lax.axis_index("c")
    base = wid * BPW

    pltpu.sync_copy(xa_hbm.at[pl.ds(base, BPW)], idx_a)
    pltpu.sync_copy(xb_hbm.at[pl.ds(base, BPW)], idx_b)

    def issue(i, s):
        pltpu.async_copy(emb_hbm.at[idx_a.at[i]], bufs_a[s], sems_a[s])
        pltpu.async_copy(emb_hbm.at[idx_b.at[i]], bufs_b[s], sems_b[s])

    def wait(s):
        pltpu.make_async_copy(emb_hbm.at[idx_a.at[0]], bufs_a[s], sems_a[s]).wait()
        pltpu.make_async_copy(emb_hbm.at[idx_b.at[0]], bufs_b[s], sems_b[s]).wait()

    def reduce_row(i, s):
        def red(buf):
            def body(j, ms):
                return tuple(
                    jnp.maximum(ms[c], buf[j, pl.ds(c * L, L)])
                    for c in range(4)
                )
            return body

        ms = tuple(jnp.full((L,), -jnp.inf, jnp.float32) for _ in range(4))
        ms = lax.fori_loop(0, SA, red(bufs_a[s]), ms, unroll=8)
        ms = lax.fori_loop(0, SB, red(bufs_b[s]), ms, unroll=8)
        for c in range(4):
            out_v[i, pl.ds(c * L, L)] = ms[c]

    for s in range(NSLOT - 1):
        issue(s, s)

    def group(p, carry):
        i0 = NSLOT * p
        for s in range(NSLOT):
            i = i0 + s
            issue(jnp.minimum(i + NSLOT - 1, BPW - 1), (s + NSLOT - 1) % NSLOT)
            wait(s)
            reduce_row(i, s)
        return carry

    lax.fori_loop(0, BPW // NSLOT, group, 0)
    for s in range(NSLOT - 1):
        wait(s)
    pltpu.sync_copy(out_v, out_hbm.at[pl.ds(base, BPW)])


def _mm_body(p_ref, w_ref, b_ref, o_ref):
    o_ref[...] = (
        lax.dot_general(
            p_ref[...], w_ref[...],
            (((1,), (1,)), ((), ())),
            preferred_element_type=jnp.float32,
        )
        + b_ref[...]
    )


_mm = pl.pallas_call(
    _mm_body,
    grid=(8,),
    in_specs=[
        pl.BlockSpec((B // 8, D), lambda i: (i, 0)),
        pl.BlockSpec((N_LOCS, D), lambda i: (0, 0)),
        pl.BlockSpec((1, N_LOCS), lambda i: (0, 0)),
    ],
    out_specs=pl.BlockSpec((B // 8, N_LOCS), lambda i: (i, 0)),
    out_shape=jax.ShapeDtypeStruct((B, N_LOCS), jnp.float32),
)


def kernel(x, emb, W_fc, b_fc):
    x = x.astype(jnp.int32)
    xr = jnp.where(x < VP, 2 * x, 2 * x - (2 * VP - 1))
    xa = xr[:, :SA]
    xb = xr[:, SA:]
    embT = emb.T
    tailT = lax.slice(embT, (0, V - TAIL), (D, V))
    pair_tab = _transposer(embT, embT, tailT)
    flat_tab = pair_tab.reshape(2 * VP, D)
    pooled = _pool_sc(xa, xb, flat_tab)
    return _mm(pooled, W_fc, b_fc.reshape(1, N_LOCS))
